# initial kernel scaffold (unmeasured)
import jax
import jax.numpy as jnp
from jax import lax
from jax.experimental import pallas as pl
from jax.experimental.pallas import tpu as pltpu

N_DEV = 16
B, SQ, D = 4, 256, 1024
H, DH = 8, 128
SKV = 1024
SCALE = 0.08838834764831843
N_CHUNKS = N_DEV
CHUNK_ROWS = (B * SQ) // N_CHUNKS


def kernel(x, Wq, Wo, K_ext, V_ext):
    def body(x_ref, wq_ref, wo_ref, k_ref, v_ref, out_ref,
             acc_ref, comm_ref, rs_send, rs_recv, ag_send, ag_recv,
             credit_sem):
        my = lax.axis_index("i")
        left = lax.rem(my + N_DEV - 1, N_DEV)
        right = lax.rem(my + 1, N_DEV)

        with jax.named_scope("compute_partial"):
            wq = wq_ref[...].astype(jnp.bfloat16)
            wo = wo_ref[...].astype(jnp.bfloat16)
            for b in range(B):
                xb = x_ref[b].astype(jnp.bfloat16)
                qb = jnp.dot(xb, wq, preferred_element_type=jnp.float32)
                cols = []
                for h in range(H):
                    qh = qb[:, h * DH:(h + 1) * DH].astype(jnp.bfloat16)
                    kh = k_ref[b, :, h, :].astype(jnp.bfloat16)
                    s = lax.dot_general(
                        qh, kh, (((1,), (1,)), ((), ())),
                        preferred_element_type=jnp.float32) * SCALE
                    m = jnp.max(s, axis=1, keepdims=True)
                    p = jnp.exp(s - m)
                    l = jnp.sum(p, axis=1, keepdims=True)
                    vh = v_ref[b, :, h, :].astype(jnp.bfloat16)
                    o = lax.dot_general(
                        p.astype(jnp.bfloat16), vh,
                        (((1,), (0,)), ((), ())),
                        preferred_element_type=jnp.float32)
                    cols.append(o / l)
                attn_b = jnp.concatenate(cols, axis=1)
                pb = jnp.dot(attn_b.astype(jnp.bfloat16), wo,
                             preferred_element_type=jnp.float32)
                for j in range(B):
                    acc_ref[B * b + j] = pb[j * CHUNK_ROWS:(j + 1) * CHUNK_ROWS, :]

        with jax.named_scope("barrier"):
            bar = pltpu.get_barrier_semaphore()
            for nbr in (left, right):
                pl.semaphore_signal(bar, inc=1, device_id=(nbr,),
                                    device_id_type=pl.DeviceIdType.MESH)
            pl.semaphore_wait(bar, 2)

        for s in range(N_DEV - 1):
            with jax.named_scope(f"rs_hop{s}"):
                c_send = lax.rem(my - s + 2 * N_DEV, N_DEV)
                rdma = pltpu.make_async_remote_copy(
                    src_ref=acc_ref.at[c_send],
                    dst_ref=comm_ref.at[s],
                    send_sem=rs_send.at[s],
                    recv_sem=rs_recv.at[s],
                    device_id=(right,),
                    device_id_type=pl.DeviceIdType.MESH,
                )
                rdma.start()
                rdma.wait()
                c_recv = lax.rem(my - s - 1 + 2 * N_DEV, N_DEV)
                acc_ref[c_recv] = acc_ref[c_recv] + comm_ref[s]

        with jax.named_scope("rs_ag_credit"):
            pl.semaphore_signal(credit_sem, inc=1, device_id=(left,),
                                device_id_type=pl.DeviceIdType.MESH)
            pl.semaphore_wait(credit_sem, 1)

        for t in range(N_DEV - 1):
            with jax.named_scope(f"ag_hop{t}"):
                c_send = lax.rem(my + 1 - t + 2 * N_DEV, N_DEV)
                rdma = pltpu.make_async_remote_copy(
                    src_ref=acc_ref.at[c_send],
                    dst_ref=acc_ref.at[c_send],
                    send_sem=ag_send.at[t],
                    recv_sem=ag_recv.at[t],
                    device_id=(right,),
                    device_id_type=pl.DeviceIdType.MESH,
                )
                rdma.start()
                rdma.wait()

        with jax.named_scope("store_out"):
            for b in range(B):
                out_ref[b] = jnp.concatenate(
                    [acc_ref[B * b + j] for j in range(B)], axis=0)

    return pl.pallas_call(
        body,
        out_shape=jax.ShapeDtypeStruct((B, SQ, D), jnp.float32),
        in_specs=[pl.BlockSpec(memory_space=pltpu.VMEM)] * 5,
        out_specs=pl.BlockSpec(memory_space=pltpu.VMEM),
        scratch_shapes=[
            pltpu.VMEM((N_CHUNKS, CHUNK_ROWS, D), jnp.float32),
            pltpu.VMEM((N_DEV - 1, CHUNK_ROWS, D), jnp.float32),
            pltpu.SemaphoreType.DMA((N_DEV - 1,)),
            pltpu.SemaphoreType.DMA((N_DEV - 1,)),
            pltpu.SemaphoreType.DMA((N_DEV - 1,)),
            pltpu.SemaphoreType.DMA((N_DEV - 1,)),
            pltpu.SemaphoreType.REGULAR,
        ],
        compiler_params=pltpu.CompilerParams(collective_id=0),
    )(x, Wq, Wo, K_ext, V_ext)


# baseline (device time: 208190 ns/iter reference)
import jax
import jax.numpy as jnp
from jax import lax
from jax.experimental import pallas as pl
from jax.experimental.pallas import tpu as pltpu

N_DEV = 16
B, SQ, D = 4, 256, 1024
H, DH = 8, 128
SKV = 1024
SCALE = 0.08838834764831843
N_CHUNKS = N_DEV
CHUNK_ROWS = (B * SQ) // N_CHUNKS


def kernel(x, Wq, Wo, K_ext, V_ext):
    def body(x_ref, wq_ref, wo_ref, k_ref, v_ref, out_ref,
             acc_ref, comm_ref, rs_send, rs_recv, ag_send, ag_recv,
             credit_sem):
        my = lax.axis_index("i")
        left = lax.rem(my + N_DEV - 1, N_DEV)
        right = lax.rem(my + 1, N_DEV)

        with jax.named_scope("compute_partial"):
            wq = wq_ref[...]
            wo = wo_ref[...]
            for b in range(B):
                xb = x_ref[b]
                qb = jnp.dot(xb, wq, preferred_element_type=jnp.float32)
                cols = []
                for h in range(H):
                    qh = qb[:, h * DH:(h + 1) * DH].astype(jnp.bfloat16)
                    kh = k_ref[b, h]
                    s = lax.dot_general(
                        qh, kh, (((1,), (1,)), ((), ())),
                        preferred_element_type=jnp.float32) * SCALE
                    m = jnp.max(s, axis=1, keepdims=True)
                    p = jnp.exp(s - m)
                    l = jnp.sum(p, axis=1, keepdims=True)
                    vh = v_ref[b, h]
                    o = lax.dot_general(
                        p.astype(jnp.bfloat16), vh,
                        (((1,), (0,)), ((), ())),
                        preferred_element_type=jnp.float32)
                    cols.append(o / l)
                attn_b = jnp.concatenate(cols, axis=1)
                pb = jnp.dot(attn_b.astype(jnp.bfloat16), wo,
                             preferred_element_type=jnp.float32)
                for j in range(B):
                    acc_ref[B * b + j] = pb[j * CHUNK_ROWS:(j + 1) * CHUNK_ROWS, :]

        with jax.named_scope("barrier"):
            bar = pltpu.get_barrier_semaphore()
            for nbr in (left, right):
                pl.semaphore_signal(bar, inc=1, device_id=(nbr,),
                                    device_id_type=pl.DeviceIdType.MESH)
            pl.semaphore_wait(bar, 2)

        for s in range(N_DEV - 1):
            with jax.named_scope(f"rs_hop{s}"):
                c_send = lax.rem(my - s + 2 * N_DEV, N_DEV)
                rdma = pltpu.make_async_remote_copy(
                    src_ref=acc_ref.at[c_send],
                    dst_ref=comm_ref.at[s],
                    send_sem=rs_send.at[s],
                    recv_sem=rs_recv.at[s],
                    device_id=(right,),
                    device_id_type=pl.DeviceIdType.MESH,
                )
                rdma.start()
                rdma.wait()
                c_recv = lax.rem(my - s - 1 + 2 * N_DEV, N_DEV)
                acc_ref[c_recv] = acc_ref[c_recv] + comm_ref[s]

        with jax.named_scope("rs_ag_credit"):
            pl.semaphore_signal(credit_sem, inc=1, device_id=(left,),
                                device_id_type=pl.DeviceIdType.MESH)
            pl.semaphore_wait(credit_sem, 1)

        for t in range(N_DEV - 1):
            with jax.named_scope(f"ag_hop{t}"):
                c_send = lax.rem(my + 1 - t + 2 * N_DEV, N_DEV)
                rdma = pltpu.make_async_remote_copy(
                    src_ref=acc_ref.at[c_send],
                    dst_ref=acc_ref.at[c_send],
                    send_sem=ag_send.at[t],
                    recv_sem=ag_recv.at[t],
                    device_id=(right,),
                    device_id_type=pl.DeviceIdType.MESH,
                )
                rdma.start()
                rdma.wait()

        with jax.named_scope("store_out"):
            for b in range(B):
                out_ref[b] = jnp.concatenate(
                    [acc_ref[B * b + j] for j in range(B)], axis=0)

    return pl.pallas_call(
        body,
        out_shape=jax.ShapeDtypeStruct((B, SQ, D), jnp.float32),
        in_specs=[pl.BlockSpec(memory_space=pltpu.VMEM)] * 5,
        out_specs=pl.BlockSpec(memory_space=pltpu.VMEM),
        scratch_shapes=[
            pltpu.VMEM((N_CHUNKS, CHUNK_ROWS, D), jnp.float32),
            pltpu.VMEM((N_DEV - 1, CHUNK_ROWS, D), jnp.float32),
            pltpu.SemaphoreType.DMA((N_DEV - 1,)),
            pltpu.SemaphoreType.DMA((N_DEV - 1,)),
            pltpu.SemaphoreType.DMA((N_DEV - 1,)),
            pltpu.SemaphoreType.DMA((N_DEV - 1,)),
            pltpu.SemaphoreType.REGULAR,
        ],
        compiler_params=pltpu.CompilerParams(
            collective_id=0,
            vmem_limit_bytes=100 * 1024 * 1024,
        ),
    )(
        x.astype(jnp.bfloat16),
        Wq.astype(jnp.bfloat16),
        Wo.astype(jnp.bfloat16),
        jnp.transpose(K_ext, (0, 2, 1, 3)).astype(jnp.bfloat16),
        jnp.transpose(V_ext, (0, 2, 1, 3)).astype(jnp.bfloat16),
    )


# device time: 113233 ns/iter; 1.8386x vs baseline; 1.8386x over previous
import jax
import jax.numpy as jnp
from jax import lax
from jax.experimental import pallas as pl
from jax.experimental.pallas import tpu as pltpu

N_DEV = 16
B, SQ, D = 4, 256, 1024
H, DH = 8, 128
SKV = 1024
SCALE = 0.08838834764831843
CHUNK_ROWS = (B * SQ) // N_DEV


def kernel(x, Wq, Wo, K_ext, V_ext):
    def body(x_ref, wq_ref, wo_ref, k_ref, v_ref, out_ref,
             acc_ref, sendbuf, rsbuf, agsend, agbuf,
             rs_send, rs_recv, ag_send, ag_recv):
        my = lax.axis_index("i")

        with jax.named_scope("compute_partial"):
            wq = wq_ref[...]
            wo = wo_ref[...]
            for b in range(B):
                xb = x_ref[b]
                qb = jnp.dot(xb, wq, preferred_element_type=jnp.float32)
                cols = []
                for h in range(H):
                    qh = qb[:, h * DH:(h + 1) * DH].astype(jnp.bfloat16)
                    kh = k_ref[b, h]
                    s = lax.dot_general(
                        qh, kh, (((1,), (1,)), ((), ())),
                        preferred_element_type=jnp.float32) * SCALE
                    m = jnp.max(s, axis=1, keepdims=True)
                    p = jnp.exp(s - m)
                    l = jnp.sum(p, axis=1, keepdims=True)
                    vh = v_ref[b, h]
                    o = lax.dot_general(
                        p.astype(jnp.bfloat16), vh,
                        (((1,), (0,)), ((), ())),
                        preferred_element_type=jnp.float32)
                    cols.append(o / l)
                attn_b = jnp.concatenate(cols, axis=1)
                pb = jnp.dot(attn_b.astype(jnp.bfloat16), wo,
                             preferred_element_type=jnp.float32)
                for j in range(B):
                    acc_ref[B * b + j] = pb[j * CHUNK_ROWS:(j + 1) * CHUNK_ROWS, :]

        with jax.named_scope("stage_send"):
            for k in range(1, N_DEV):
                t = lax.rem(my + k, N_DEV)
                sendbuf[k - 1] = acc_ref[t].astype(jnp.bfloat16)

        with jax.named_scope("barrier"):
            bar = pltpu.get_barrier_semaphore()
            for k in range(1, N_DEV):
                t = lax.rem(my + k, N_DEV)
                pl.semaphore_signal(bar, inc=1, device_id=(t,),
                                    device_id_type=pl.DeviceIdType.MESH)
            pl.semaphore_wait(bar, N_DEV - 1)

        rs = []
        with jax.named_scope("rs_send"):
            for k in range(1, N_DEV):
                t = lax.rem(my + k, N_DEV)
                rdma = pltpu.make_async_remote_copy(
                    src_ref=sendbuf.at[k - 1],
                    dst_ref=rsbuf.at[k - 1],
                    send_sem=rs_send.at[k - 1],
                    recv_sem=rs_recv.at[k - 1],
                    device_id=(t,),
                    device_id_type=pl.DeviceIdType.MESH,
                )
                rdma.start()
                rs.append(rdma)

        with jax.named_scope("rs_wait_reduce"):
            for r in rs:
                r.wait_recv()
            red = acc_ref[my]
            for k in range(1, N_DEV):
                red = red + rsbuf[k - 1].astype(jnp.float32)
            acc_ref[my] = red
            agsend[...] = red.astype(jnp.bfloat16)

        ag = []
        with jax.named_scope("ag_send"):
            for k in range(1, N_DEV):
                t = lax.rem(my + k, N_DEV)
                rdma = pltpu.make_async_remote_copy(
                    src_ref=agsend,
                    dst_ref=agbuf.at[k - 1],
                    send_sem=ag_send.at[k - 1],
                    recv_sem=ag_recv.at[k - 1],
                    device_id=(t,),
                    device_id_type=pl.DeviceIdType.MESH,
                )
                rdma.start()
                ag.append(rdma)

        with jax.named_scope("ag_wait_store"):
            for k in range(1, N_DEV):
                ag[k - 1].wait_recv()
                c = lax.rem(my - k + 2 * N_DEV, N_DEV)
                acc_ref[c] = agbuf[k - 1].astype(jnp.float32)

        with jax.named_scope("store_out"):
            for b in range(B):
                out_ref[b] = jnp.concatenate(
                    [acc_ref[B * b + j] for j in range(B)], axis=0)

        with jax.named_scope("drain_sends"):
            for r in rs + ag:
                r.wait_send()

    return pl.pallas_call(
        body,
        out_shape=jax.ShapeDtypeStruct((B, SQ, D), jnp.float32),
        in_specs=[pl.BlockSpec(memory_space=pltpu.VMEM)] * 5,
        out_specs=pl.BlockSpec(memory_space=pltpu.VMEM),
        scratch_shapes=[
            pltpu.VMEM((N_DEV, CHUNK_ROWS, D), jnp.float32),
            pltpu.VMEM((N_DEV - 1, CHUNK_ROWS, D), jnp.bfloat16),
            pltpu.VMEM((N_DEV - 1, CHUNK_ROWS, D), jnp.bfloat16),
            pltpu.VMEM((CHUNK_ROWS, D), jnp.bfloat16),
            pltpu.VMEM((N_DEV - 1, CHUNK_ROWS, D), jnp.bfloat16),
            pltpu.SemaphoreType.DMA((N_DEV - 1,)),
            pltpu.SemaphoreType.DMA((N_DEV - 1,)),
            pltpu.SemaphoreType.DMA((N_DEV - 1,)),
            pltpu.SemaphoreType.DMA((N_DEV - 1,)),
        ],
        compiler_params=pltpu.CompilerParams(
            collective_id=0,
            vmem_limit_bytes=100 * 1024 * 1024,
        ),
    )(
        x.astype(jnp.bfloat16),
        Wq.astype(jnp.bfloat16),
        Wo.astype(jnp.bfloat16),
        jnp.transpose(K_ext, (0, 2, 1, 3)).astype(jnp.bfloat16),
        jnp.transpose(V_ext, (0, 2, 1, 3)).astype(jnp.bfloat16),
    )


# device time: 112199 ns/iter; 1.8555x vs baseline; 1.0092x over previous
import jax
import jax.numpy as jnp
from jax import lax
from jax.experimental import pallas as pl
from jax.experimental.pallas import tpu as pltpu

N_DEV = 16
B, SQ, D = 4, 256, 1024
H, DH = 8, 128
SKV = 1024
SCALE = 0.08838834764831843
CHUNK_ROWS = (B * SQ) // N_DEV


def kernel(x, Wq, Wo, K_ext, V_ext):
    def body(x_ref, wq_ref, wo_ref, k_ref, v_ref, out_ref,
             acc_ref, sendbuf, rsbuf, agsend, agbuf,
             rs_send, rs_recv, ag_send, ag_recv):
        my = lax.axis_index("i")

        with jax.named_scope("barrier"):
            bar = pltpu.get_barrier_semaphore()
            for k in range(1, N_DEV):
                t = lax.rem(my + k, N_DEV)
                pl.semaphore_signal(bar, inc=1, device_id=(t,),
                                    device_id_type=pl.DeviceIdType.MESH)
            pl.semaphore_wait(bar, N_DEV - 1)

        wq = wq_ref[...]
        wo = wo_ref[...]
        for b in range(B):
            with jax.named_scope(f"compute_b{b}"):
                xb = x_ref[b]
                qb = jnp.dot(xb, wq, preferred_element_type=jnp.float32)
                cols = []
                for h in range(H):
                    qh = qb[:, h * DH:(h + 1) * DH].astype(jnp.bfloat16)
                    kh = k_ref[b, h]
                    s = lax.dot_general(
                        qh, kh, (((1,), (1,)), ((), ())),
                        preferred_element_type=jnp.float32) * SCALE
                    m = jnp.max(s, axis=1, keepdims=True)
                    p = jnp.exp(s - m)
                    l = jnp.sum(p, axis=1, keepdims=True)
                    vh = v_ref[b, h]
                    o = lax.dot_general(
                        p.astype(jnp.bfloat16), vh,
                        (((1,), (0,)), ((), ())),
                        preferred_element_type=jnp.float32)
                    cols.append(o / l)
                attn_b = jnp.concatenate(cols, axis=1)
                pb = jnp.dot(attn_b.astype(jnp.bfloat16), wo,
                             preferred_element_type=jnp.float32)
            with jax.named_scope(f"rs_send_b{b}"):
                for j in range(B):
                    c = B * b + j
                    pc = pb[j * CHUNK_ROWS:(j + 1) * CHUNK_ROWS, :]
                    acc_ref[c] = pc
                    k = lax.rem(c - my + N_DEV, N_DEV)
                    kk = jnp.maximum(k, 1) - 1
                    sendbuf[kk] = pc.astype(jnp.bfloat16)

                    @pl.when(k != 0)
                    def _():
                        rdma = pltpu.make_async_remote_copy(
                            src_ref=sendbuf.at[kk],
                            dst_ref=rsbuf.at[kk],
                            send_sem=rs_send.at[kk],
                            recv_sem=rs_recv.at[kk],
                            device_id=(c,),
                            device_id_type=pl.DeviceIdType.MESH,
                        )
                        rdma.start()

        with jax.named_scope("rs_wait_reduce"):
            for i in range(N_DEV - 1):
                pltpu.make_async_remote_copy(
                    src_ref=sendbuf.at[i],
                    dst_ref=rsbuf.at[i],
                    send_sem=rs_send.at[i],
                    recv_sem=rs_recv.at[i],
                    device_id=(my,),
                    device_id_type=pl.DeviceIdType.MESH,
                ).wait_recv()
            red = acc_ref[my]
            for i in range(N_DEV - 1):
                red = red + rsbuf[i].astype(jnp.float32)
            acc_ref[my] = red
            agsend[...] = red.astype(jnp.bfloat16)

        ag = []
        with jax.named_scope("ag_send"):
            for k in range(1, N_DEV):
                t = lax.rem(my + k, N_DEV)
                rdma = pltpu.make_async_remote_copy(
                    src_ref=agsend,
                    dst_ref=agbuf.at[k - 1],
                    send_sem=ag_send.at[k - 1],
                    recv_sem=ag_recv.at[k - 1],
                    device_id=(t,),
                    device_id_type=pl.DeviceIdType.MESH,
                )
                rdma.start()
                ag.append(rdma)

        with jax.named_scope("ag_wait_store"):
            for k in range(1, N_DEV):
                ag[k - 1].wait_recv()
                c = lax.rem(my - k + 2 * N_DEV, N_DEV)
                acc_ref[c] = agbuf[k - 1].astype(jnp.float32)

        with jax.named_scope("store_out"):
            for b in range(B):
                out_ref[b] = jnp.concatenate(
                    [acc_ref[B * b + j] for j in range(B)], axis=0)

        with jax.named_scope("drain_sends"):
            for i in range(N_DEV - 1):
                pltpu.make_async_remote_copy(
                    src_ref=sendbuf.at[i],
                    dst_ref=rsbuf.at[i],
                    send_sem=rs_send.at[i],
                    recv_sem=rs_recv.at[i],
                    device_id=(my,),
                    device_id_type=pl.DeviceIdType.MESH,
                ).wait_send()
            for r in ag:
                r.wait_send()

    return pl.pallas_call(
        body,
        out_shape=jax.ShapeDtypeStruct((B, SQ, D), jnp.float32),
        in_specs=[pl.BlockSpec(memory_space=pltpu.VMEM)] * 5,
        out_specs=pl.BlockSpec(memory_space=pltpu.VMEM),
        scratch_shapes=[
            pltpu.VMEM((N_DEV, CHUNK_ROWS, D), jnp.float32),
            pltpu.VMEM((N_DEV - 1, CHUNK_ROWS, D), jnp.bfloat16),
            pltpu.VMEM((N_DEV - 1, CHUNK_ROWS, D), jnp.bfloat16),
            pltpu.VMEM((CHUNK_ROWS, D), jnp.bfloat16),
            pltpu.VMEM((N_DEV - 1, CHUNK_ROWS, D), jnp.bfloat16),
            pltpu.SemaphoreType.DMA((N_DEV - 1,)),
            pltpu.SemaphoreType.DMA((N_DEV - 1,)),
            pltpu.SemaphoreType.DMA((N_DEV - 1,)),
            pltpu.SemaphoreType.DMA((N_DEV - 1,)),
        ],
        compiler_params=pltpu.CompilerParams(
            collective_id=0,
            vmem_limit_bytes=100 * 1024 * 1024,
        ),
    )(
        x.astype(jnp.bfloat16),
        Wq.astype(jnp.bfloat16),
        Wo.astype(jnp.bfloat16),
        jnp.transpose(K_ext, (0, 2, 1, 3)).astype(jnp.bfloat16),
        jnp.transpose(V_ext, (0, 2, 1, 3)).astype(jnp.bfloat16),
    )
